# baseline (speedup 1.0000x reference)
import functools

import jax
import jax.numpy as jnp
from jax import lax
from jax.experimental import pallas as pl
from jax.experimental.pallas import tpu as pltpu
from jax.experimental.pallas import tpu_sc as plsc

VOCAB = 1000000
B = 16384
H = 200
D = 32
C = 16
HALF = 100
NHALF = B * 2

NC = 2
NS = 16
NW = NC * NS

HW_PER_W = NHALF // NW
ROWS_PER_W = B // NW
CH_H = 128
NCHUNK = HW_PER_W // CH_H
G = 8


VOCAB = 1000000
TP_PER_W = 31248
TP_CH = 768
TP_NCH = TP_PER_W // TP_CH
TP_TAIL = TP_PER_W - TP_NCH * TP_CH
TP_REST = VOCAB - TP_PER_W * NW


def _sc_body(x2_hbm, tabT_hbm, out_hbm, lin_hbm, idx_v, rows_v, out_v,
             tbuf_v, wbuf_v, isems, gsems, csem):
  cid = lax.axis_index("c")
  sid = lax.axis_index("s")
  wid = sid * NC + cid
  hbase0 = wid * HW_PER_W
  obase = wid * ROWS_PER_W

  zero = jnp.zeros((16,), jnp.float32)

  lane = lax.iota(jnp.int32, 16)

  def tp_chunk(cb, n):
    pltpu.sync_copy(tabT_hbm.at[:, pl.ds(cb, n)], tbuf_v.at[:, pl.ds(0, n)])

    def tp_row(r, _):
      col = jnp.full((16,), r, jnp.int32)
      g0 = plsc.load_gather(tbuf_v, [lane, col])
      g1 = plsc.load_gather(tbuf_v, [lane + 16, col])
      wbuf_v[r, 0:16] = g0
      wbuf_v[r, 16:32] = g1
      return _

    lax.fori_loop(0, n, tp_row, None)
    pltpu.sync_copy(wbuf_v.at[pl.ds(0, n), :], lin_hbm.at[pl.ds(cb, n), :])

  tpbase = wid * TP_PER_W

  def tp_body(k, _):
    tp_chunk(tpbase + k * TP_CH, TP_CH)
    return _

  lax.fori_loop(0, TP_NCH, tp_body, None)
  tp_chunk(tpbase + TP_NCH * TP_CH, TP_TAIL)

  @pl.when(wid == NW - 1)
  def _rest():
    tp_chunk(TP_PER_W * NW, TP_REST)

  plsc.subcore_barrier()
  pltpu.core_barrier(csem, core_axis_name="c")
  tab_hbm = lin_hbm

  def fire_idx(c, buf):
    pltpu.async_copy(x2_hbm.at[pl.ds(hbase0 + c * CH_H, CH_H), :],
                     idx_v.at[buf], isems.at[buf])

  def wait_idx(buf):
    pltpu.make_async_copy(x2_hbm.at[pl.ds(hbase0, CH_H), :], idx_v.at[buf],
                          isems.at[buf]).wait()

  def fire_gather(p, h, g):
    pltpu.async_copy(tab_hbm.at[idx_v.at[p, h]], rows_v.at[g], gsems.at[g])

  def wait_gather(g):
    pltpu.make_async_copy(tab_hbm.at[pl.ds(0, HALF), :], rows_v.at[g],
                          gsems.at[g]).wait()

  def reduce_store(c, hb, g):
    def red_body(j, acc):
      acc = list(acc)
      for k in range(4):
        jj = j * 4 + k
        acc[2 * k] = acc[2 * k] + rows_v[g, jj, 0:16]
        acc[2 * k + 1] = acc[2 * k + 1] + rows_v[g, jj, 16:32]
      return tuple(acc)

    acc = lax.fori_loop(0, HALF // 4, red_body, (zero,) * 8)
    b0 = (acc[0] + acc[2]) + (acc[4] + acc[6])
    b1 = (acc[1] + acc[3]) + (acc[5] + acc[7])
    slot = c * (CH_H // 2) + hb // 2 + g // 2
    if g % 2 == 0:
      out_v[slot, 0:16] = b0
      out_v[slot, 16:32] = b1
    else:
      out_v[slot, 0:16] = out_v[slot, 0:16] + b0
      out_v[slot, 16:32] = out_v[slot, 16:32] + b1

  fire_idx(0, 0)

  def pair_body(pair, _):
    for p in range(2):
      c = pair * 2 + p
      wait_idx(p)

      @pl.when(c + 1 < NCHUNK)
      def _prefetch():
        fire_idx(c + 1, 1 - p)

      for g in range(G):
        fire_gather(p, g, g)

      def ring_body(i, _):
        hb = i * G
        for g in range(G):
          wait_gather(g)
          reduce_store(c, hb, g)
          fire_gather(p, hb + G + g, g)
        return _

      lax.fori_loop(0, CH_H // G - 1, ring_body, None)
      for g in range(G):
        wait_gather(g)
        reduce_store(c, CH_H - G, g)
    return _

  lax.fori_loop(0, NCHUNK // 2, pair_body, None)
  pltpu.sync_copy(out_v, out_hbm.at[pl.ds(obase, ROWS_PER_W), :])


@jax.jit
def _sc_pool(x2, embedT):
  mesh = plsc.VectorSubcoreMesh(
      core_axis_name="c", subcore_axis_name="s", num_cores=NC,
      num_subcores=NS)
  f = pl.kernel(
      _sc_body,
      out_type=jax.ShapeDtypeStruct((B, D), jnp.float32),
      mesh=mesh,
      scratch_types=[
          pltpu.HBM((VOCAB, D), jnp.float32),
          pltpu.VMEM((2, CH_H, HALF), jnp.int32),
          pltpu.VMEM((G, HALF, D), jnp.float32),
          pltpu.VMEM((ROWS_PER_W, D), jnp.float32),
          pltpu.VMEM((D, TP_CH), jnp.float32),
          pltpu.VMEM((TP_CH, D), jnp.float32),
          pltpu.SemaphoreType.DMA((2,)),
          pltpu.SemaphoreType.DMA((G,)),
          pltpu.SemaphoreType.REGULAR,
      ],
      compiler_params=pltpu.CompilerParams(use_tc_tiling_on_sc=False, needs_layout_passes=False),
  )
  return f(x2, embedT)


def _tc_body(ms_ref, wt_ref, b_ref, out_ref):
  m = ms_ref[...] * jnp.float32(1.0 / H)
  logits = jnp.dot(m, wt_ref[...], preferred_element_type=jnp.float32)
  logits = logits + b_ref[...]
  mx = jnp.max(logits, axis=1, keepdims=True)
  s = logits - mx
  lse = jnp.log(jnp.sum(jnp.exp(s), axis=1, keepdims=True))
  out_ref[...] = s - lse


@jax.jit
def _tc_head(msum, wt, b2):
  blk = 2048
  return pl.pallas_call(
      _tc_body,
      grid=(B // blk,),
      in_specs=[
          pl.BlockSpec((blk, D), lambda i: (i, 0)),
          pl.BlockSpec((D, C), lambda i: (0, 0)),
          pl.BlockSpec((1, C), lambda i: (0, 0)),
      ],
      out_specs=pl.BlockSpec((blk, C), lambda i: (i, 0)),
      out_shape=jax.ShapeDtypeStruct((B, C), jnp.float32),
  )(msum, wt, b2)


def kernel(x, embed, fc_w, fc_b):
  x2 = x.astype(jnp.int32).reshape(NHALF, HALF)
  msum = _sc_pool(x2, embed.T)
  return _tc_head(msum, fc_w.T, fc_b.reshape(1, C))


# speedup vs baseline: 4.8430x; 4.8430x over previous
import functools

import jax
import jax.numpy as jnp
from jax import lax
from jax.experimental import pallas as pl
from jax.experimental.pallas import tpu as pltpu
from jax.experimental.pallas import tpu_sc as plsc

VOCAB = 1000000
B = 16384
H = 200
D = 32
C = 16
HALF = 100
NHALF = B * 2

NC = 2
NS = 16
NW = NC * NS

HW_PER_W = NHALF // NW
ROWS_PER_W = B // NW
CH_H = 256
NCHUNK = HW_PER_W // CH_H
G = 16


def _sc_body(x2_hbm, tab_hbm, out_hbm, idx_v, rows_v, out_v, isems, gsems):
  cid = lax.axis_index("c")
  sid = lax.axis_index("s")
  wid = sid * NC + cid
  hbase0 = wid * HW_PER_W
  obase = wid * ROWS_PER_W

  zero = jnp.zeros((16,), jnp.float32)

  def fire_idx(c, buf):
    pltpu.async_copy(x2_hbm.at[pl.ds(hbase0 + c * CH_H, CH_H), :],
                     idx_v.at[buf], isems.at[buf])

  def wait_idx(buf):
    pltpu.make_async_copy(x2_hbm.at[pl.ds(hbase0, CH_H), :], idx_v.at[buf],
                          isems.at[buf]).wait()

  def fire_gather(p, h, g):
    pltpu.async_copy(tab_hbm.at[idx_v.at[p, h]], rows_v.at[g], gsems.at[g])

  def wait_gather(g):
    pltpu.make_async_copy(tab_hbm.at[pl.ds(0, HALF), :], rows_v.at[g],
                          gsems.at[g]).wait()

  def reduce_store(c, hb, g):
    def red_body(j, acc):
      acc = list(acc)
      for k in range(4):
        jj = j * 4 + k
        acc[2 * k] = acc[2 * k] + rows_v[g, jj, 0:16]
        acc[2 * k + 1] = acc[2 * k + 1] + rows_v[g, jj, 16:32]
      return tuple(acc)

    acc = lax.fori_loop(0, HALF // 4, red_body, (zero,) * 8)
    b0 = (acc[0] + acc[2]) + (acc[4] + acc[6])
    b1 = (acc[1] + acc[3]) + (acc[5] + acc[7])
    slot = c * (CH_H // 2) + hb // 2 + g // 2
    if g % 2 == 0:
      out_v[slot, 0:16] = b0
      out_v[slot, 16:32] = b1
    else:
      out_v[slot, 0:16] = out_v[slot, 0:16] + b0
      out_v[slot, 16:32] = out_v[slot, 16:32] + b1

  fire_idx(0, 0)

  def pair_body(pair, _):
    for p in range(2):
      c = pair * 2 + p
      wait_idx(p)

      @pl.when(c + 1 < NCHUNK)
      def _prefetch():
        fire_idx(c + 1, 1 - p)

      for g in range(G):
        fire_gather(p, g, g)

      def ring_body(i, _):
        hb = i * G
        for g in range(G):
          wait_gather(g)
          reduce_store(c, hb, g)
          fire_gather(p, hb + G + g, g)
        return _

      lax.fori_loop(0, CH_H // G - 1, ring_body, None)
      for g in range(G):
        wait_gather(g)
        reduce_store(c, CH_H - G, g)
    return _

  lax.fori_loop(0, NCHUNK // 2, pair_body, None)
  pltpu.sync_copy(out_v, out_hbm.at[pl.ds(obase, ROWS_PER_W), :])


@jax.jit
def _sc_pool(x2, embed):
  mesh = plsc.VectorSubcoreMesh(
      core_axis_name="c", subcore_axis_name="s", num_cores=NC,
      num_subcores=NS)
  f = pl.kernel(
      _sc_body,
      out_type=jax.ShapeDtypeStruct((B, D), jnp.float32),
      mesh=mesh,
      scratch_types=[
          pltpu.VMEM((2, CH_H, HALF), jnp.int32),
          pltpu.VMEM((G, HALF, D), jnp.float32),
          pltpu.VMEM((ROWS_PER_W, D), jnp.float32),
          pltpu.SemaphoreType.DMA((2,)),
          pltpu.SemaphoreType.DMA((G,)),
      ],
      compiler_params=pltpu.CompilerParams(use_tc_tiling_on_sc=False),
  )
  return f(x2, embed)


def _tc_body(ms_ref, wt_ref, b_ref, out_ref):
  m = ms_ref[...] * jnp.float32(1.0 / H)
  logits = jnp.dot(m, wt_ref[...], preferred_element_type=jnp.float32)
  logits = logits + b_ref[...]
  mx = jnp.max(logits, axis=1, keepdims=True)
  s = logits - mx
  lse = jnp.log(jnp.sum(jnp.exp(s), axis=1, keepdims=True))
  out_ref[...] = s - lse


@jax.jit
def _tc_head(msum, wt, b2):
  blk = 2048
  return pl.pallas_call(
      _tc_body,
      grid=(B // blk,),
      in_specs=[
          pl.BlockSpec((blk, D), lambda i: (i, 0)),
          pl.BlockSpec((D, C), lambda i: (0, 0)),
          pl.BlockSpec((1, C), lambda i: (0, 0)),
      ],
      out_specs=pl.BlockSpec((blk, C), lambda i: (i, 0)),
      out_shape=jax.ShapeDtypeStruct((B, C), jnp.float32),
  )(msum, wt, b2)


def kernel(x, embed, fc_w, fc_b):
  x2 = x.astype(jnp.int32).reshape(NHALF, HALF)
  msum = _sc_pool(x2, embed)
  return _tc_head(msum, fc_w.T, fc_b.reshape(1, C))


# speedup vs baseline: 4.9308x; 1.0181x over previous
import functools

import jax
import jax.numpy as jnp
from jax import lax
from jax.experimental import pallas as pl
from jax.experimental.pallas import tpu as pltpu
from jax.experimental.pallas import tpu_sc as plsc

VOCAB = 1000000
B = 16384
H = 200
D = 32
C = 16
HALF = 100
NHALF = B * 2

NC = 2
NS = 16
NW = NC * NS

HW_PER_W = NHALF // NW
ROWS_PER_W = B // NW
CH_H = 128
NCHUNK = HW_PER_W // CH_H
G = 8


def _sc_body(x2_hbm, tab_hbm, out_hbm, idx_v, rows_v, out_v, isems, gsems):
  cid = lax.axis_index("c")
  sid = lax.axis_index("s")
  wid = sid * NC + cid
  hbase0 = wid * HW_PER_W
  obase = wid * ROWS_PER_W

  zero = jnp.zeros((16,), jnp.float32)

  def fire_idx(c, buf):
    pltpu.async_copy(x2_hbm.at[pl.ds(hbase0 + c * CH_H, CH_H), :],
                     idx_v.at[buf], isems.at[buf])

  def wait_idx(buf):
    pltpu.make_async_copy(x2_hbm.at[pl.ds(hbase0, CH_H), :], idx_v.at[buf],
                          isems.at[buf]).wait()

  def fire_gather(p, h, g):
    pltpu.async_copy(tab_hbm.at[idx_v.at[p, h]], rows_v.at[g], gsems.at[g])

  def wait_gather(g):
    pltpu.make_async_copy(tab_hbm.at[pl.ds(0, HALF), :], rows_v.at[g],
                          gsems.at[g]).wait()

  def reduce_store(c, hb, g):
    def red_body(j, acc):
      acc = list(acc)
      for k in range(4):
        jj = j * 4 + k
        acc[2 * k] = acc[2 * k] + rows_v[g, jj, 0:16]
        acc[2 * k + 1] = acc[2 * k + 1] + rows_v[g, jj, 16:32]
      return tuple(acc)

    acc = lax.fori_loop(0, HALF // 4, red_body, (zero,) * 8)
    b0 = (acc[0] + acc[2]) + (acc[4] + acc[6])
    b1 = (acc[1] + acc[3]) + (acc[5] + acc[7])
    slot = c * (CH_H // 2) + hb // 2 + g // 2
    if g % 2 == 0:
      out_v[slot, 0:16] = b0
      out_v[slot, 16:32] = b1
    else:
      out_v[slot, 0:16] = out_v[slot, 0:16] + b0
      out_v[slot, 16:32] = out_v[slot, 16:32] + b1

  fire_idx(0, 0)

  def pair_body(pair, _):
    for p in range(2):
      c = pair * 2 + p
      wait_idx(p)

      @pl.when(c + 1 < NCHUNK)
      def _prefetch():
        fire_idx(c + 1, 1 - p)

      for g in range(G):
        fire_gather(p, g, g)

      def ring_body(i, _):
        hb = i * G
        for g in range(G):
          wait_gather(g)
          reduce_store(c, hb, g)
          fire_gather(p, hb + G + g, g)
        return _

      lax.fori_loop(0, CH_H // G - 1, ring_body, None)
      for g in range(G):
        wait_gather(g)
        reduce_store(c, CH_H - G, g)
    return _

  lax.fori_loop(0, NCHUNK // 2, pair_body, None)
  pltpu.sync_copy(out_v, out_hbm.at[pl.ds(obase, ROWS_PER_W), :])


@jax.jit
def _sc_pool(x2, embed):
  mesh = plsc.VectorSubcoreMesh(
      core_axis_name="c", subcore_axis_name="s", num_cores=NC,
      num_subcores=NS)
  f = pl.kernel(
      _sc_body,
      out_type=jax.ShapeDtypeStruct((B, D), jnp.float32),
      mesh=mesh,
      scratch_types=[
          pltpu.VMEM((2, CH_H, HALF), jnp.int32),
          pltpu.VMEM((G, HALF, D), jnp.float32),
          pltpu.VMEM((ROWS_PER_W, D), jnp.float32),
          pltpu.SemaphoreType.DMA((2,)),
          pltpu.SemaphoreType.DMA((G,)),
      ],
      compiler_params=pltpu.CompilerParams(use_tc_tiling_on_sc=False),
  )
  return f(x2, embed)


def _tc_body(ms_ref, wt_ref, b_ref, out_ref):
  m = ms_ref[...] * jnp.float32(1.0 / H)
  logits = jnp.dot(m, wt_ref[...], preferred_element_type=jnp.float32)
  logits = logits + b_ref[...]
  mx = jnp.max(logits, axis=1, keepdims=True)
  s = logits - mx
  lse = jnp.log(jnp.sum(jnp.exp(s), axis=1, keepdims=True))
  out_ref[...] = s - lse


@jax.jit
def _tc_head(msum, wt, b2):
  blk = 2048
  return pl.pallas_call(
      _tc_body,
      grid=(B // blk,),
      in_specs=[
          pl.BlockSpec((blk, D), lambda i: (i, 0)),
          pl.BlockSpec((D, C), lambda i: (0, 0)),
          pl.BlockSpec((1, C), lambda i: (0, 0)),
      ],
      out_specs=pl.BlockSpec((blk, C), lambda i: (i, 0)),
      out_shape=jax.ShapeDtypeStruct((B, C), jnp.float32),
  )(msum, wt, b2)


def kernel(x, embed, fc_w, fc_b):
  x2 = x.astype(jnp.int32).reshape(NHALF, HALF)
  msum = _sc_pool(x2, embed)
  return _tc_head(msum, fc_w.T, fc_b.reshape(1, C))
